# baseline (speedup 1.0000x reference)
import jax
import jax.numpy as jnp
import numpy as np
from jax import lax
from jax.experimental import pallas as pl
from jax.experimental.pallas import tpu as pltpu
from jax.experimental.pallas import tpu_sc as plsc

B = 4096
N = 200
D = 128
DP = D // 2
NC = 2
NS = 16
L = 16
NW = NC * NS
UPW = B // NW
NG = 13
C1 = 128
C2 = N - C1
NEG_INF = float("-inf")
HI_MASK = np.int32(np.uint32(0xFFFF0000))


def _bf16_split(w):
    even = plsc.bitcast(w << jnp.int32(16), jnp.float32)
    odd = plsc.bitcast(w & HI_MASK, jnp.float32)
    return even, odd


def _sc_body(user_id_hbm, neg_flat_hbm, user_tab_hbm, item_tab_hbm,
             scores_out_hbm, sel_out_hbm,
             uid_v, uprows, ids_all, rows0, rows1, scores_all, selid_v,
             sem_u, s0a, s0b, s1a, s1b):
    wid = lax.axis_index("s") * NC + lax.axis_index("c")
    base_u = wid * UPW

    pltpu.sync_copy(user_id_hbm.at[pl.ds(base_u, UPW)], uid_v)
    cu = pltpu.async_copy(user_tab_hbm.at[uid_v], uprows, sem_u)
    pltpu.sync_copy(neg_flat_hbm.at[pl.ds(base_u * N, UPW * N)],
                    ids_all.at[pl.ds(0, UPW * N)])
    cu.wait()

    rows_bufs = (rows0, rows1)
    sems = ((s0a, s0b), (s1a, s1b))

    def issue_gather(u, phase):
        rb = rows_bufs[phase]
        sa, sb = sems[phase]
        pltpu.async_copy(item_tab_hbm.at[ids_all.at[pl.ds(u * N, C1)]],
                         rb.at[pl.ds(0, C1)], sa)
        pltpu.async_copy(item_tab_hbm.at[ids_all.at[pl.ds(u * N + C1, C2)]],
                         rb.at[pl.ds(C1, C2)], sb)

    def wait_gather(u, phase):
        rb = rows_bufs[phase]
        sa, sb = sems[phase]
        pltpu.make_async_copy(item_tab_hbm.at[ids_all.at[pl.ds(u * N, C1)]],
                              rb.at[pl.ds(0, C1)], sa).wait()
        pltpu.make_async_copy(item_tab_hbm.at[ids_all.at[pl.ds(u * N + C1, C2)]],
                              rb.at[pl.ds(C1, C2)], sb).wait()

    issue_gather(0, 0)

    lanes = lax.iota(jnp.int32, L)
    n_vecs = [lanes + jnp.int32(g * L) for g in range(NG)]
    lane_lt8 = lanes < 8

    def do_user(u, phase):
        rows_v = rows_bufs[phase]
        wait_gather(u, phase)

        @pl.when(u + 1 < UPW)
        def _():
            issue_gather(u + 1, 1 - phase)

        u_splat = jnp.full((L,), u, jnp.int32)

        def d_body(dp, accs):
            dp_splat = jnp.full((L,), dp, jnp.int32)
            uw = plsc.load_gather(uprows, [u_splat, dp_splat])
            ue, uo = _bf16_split(uw)
            out = []
            for g, acc in enumerate(accs):
                w = plsc.load_gather(rows_v, [n_vecs[g], dp_splat])
                e, o = _bf16_split(w)
                out.append(acc + e * ue + o * uo)
            return tuple(out)

        accs = lax.fori_loop(
            0, DP, d_body, tuple(jnp.zeros((L,), jnp.float32) for _ in range(NG)))

        cur_max = jnp.full((L,), NEG_INF)
        cur_n = jnp.full((L,), 2147483647, jnp.int32)
        cur_id = jnp.zeros((L,), jnp.int32)
        for g in range(NG):
            s_g = accs[g]
            if g == NG - 1:
                s_g = jnp.where(lane_lt8, s_g, NEG_INF)
            ids_g = ids_all[pl.ds(u * N + g * L, L)]
            upd = s_g > cur_max
            cur_max = jnp.where(upd, s_g, cur_max)
            cur_n = jnp.where(upd, n_vecs[g], cur_n)
            cur_id = jnp.where(upd, ids_g, cur_id)
            scores_all[pl.ds(u * N + g * L, L)] = s_g

        m = jnp.max(cur_max)
        big = jnp.int32(2147483647)
        n_sel = jnp.min(jnp.where(cur_max == m, cur_n, big))
        id_sel = jnp.min(jnp.where(cur_n == n_sel, cur_id, big))
        plsc.store_scatter(selid_v, [u_splat],
                           jnp.full((L,), id_sel, jnp.int32),
                           mask=lanes == 0)

    def pair_body(i, _):
        do_user(2 * i, 0)
        do_user(2 * i + 1, 1)
        return _

    lax.fori_loop(0, UPW // 2, pair_body, None)

    pltpu.sync_copy(scores_all.at[pl.ds(0, UPW * N)],
                    scores_out_hbm.at[pl.ds(base_u * N, UPW * N)])
    pltpu.sync_copy(selid_v, sel_out_hbm.at[pl.ds(base_u, UPW)])


@jax.jit
def _run(user_id, neg_flat, user_packed, item_packed):
    mesh = plsc.VectorSubcoreMesh(core_axis_name="c", subcore_axis_name="s",
                                  num_cores=NC, num_subcores=NS)
    f = pl.kernel(
        _sc_body,
        out_type=(
            jax.ShapeDtypeStruct((B * N,), jnp.float32),
            jax.ShapeDtypeStruct((B,), jnp.int32),
        ),
        mesh=mesh,
        compiler_params=pltpu.CompilerParams(needs_layout_passes=False,
                                             use_tc_tiling_on_sc=False),
        scratch_types=(
            pltpu.VMEM((UPW,), jnp.int32),
            pltpu.VMEM((UPW, DP), jnp.int32),
            pltpu.VMEM((UPW * N + L,), jnp.int32),
            pltpu.VMEM((NG * L, DP), jnp.int32),
            pltpu.VMEM((NG * L, DP), jnp.int32),
            pltpu.VMEM((UPW * N + L,), jnp.float32),
            pltpu.VMEM((UPW,), jnp.int32),
            pltpu.SemaphoreType.DMA,
            pltpu.SemaphoreType.DMA,
            pltpu.SemaphoreType.DMA,
            pltpu.SemaphoreType.DMA,
            pltpu.SemaphoreType.DMA,
        ),
    )
    return f(user_id, neg_flat, user_packed, item_packed)


def _pack_bf16(table):
    vocab = table.shape[0]
    return lax.bitcast_convert_type(
        table.astype(jnp.bfloat16).reshape(vocab, DP, 2), jnp.int32)


def kernel(user_id, neg_item_ids, user_emb_table, item_emb_table):
    scores_flat, sel = _run(user_id.astype(jnp.int32),
                            neg_item_ids.reshape(-1),
                            _pack_bf16(user_emb_table),
                            _pack_bf16(item_emb_table))
    return scores_flat.reshape(B, N), sel.reshape(B, 1)


# speedup vs baseline: 1.4924x; 1.4924x over previous
import jax
import jax.numpy as jnp
import numpy as np
from jax import lax
from jax.experimental import pallas as pl
from jax.experimental.pallas import tpu as pltpu
from jax.experimental.pallas import tpu_sc as plsc

B = 4096
N = 200
D = 128
DP = D // 2
DPP = DP + 1
NC = 2
NS = 16
L = 16
NW = NC * NS
UPW = B // NW
NG = 13
C1 = 128
C2 = N - C1
NEG_INF = float("-inf")
HI_MASK = np.int32(np.uint32(0xFFFF0000))


def _bf16_split(w):
    even = plsc.bitcast(w << jnp.int32(16), jnp.float32)
    odd = plsc.bitcast(w & HI_MASK, jnp.float32)
    return even, odd


def _sc_body(user_id_hbm, neg_flat_hbm, user_tab_hbm, item_tab_hbm,
             scores_out_hbm, sel_out_hbm,
             uid_v, uprows, ids_all, rows0, rows1, scores_all, selid_v,
             sem_u, s0a, s0b, s1a, s1b):
    wid = lax.axis_index("s") * NC + lax.axis_index("c")
    base_u = wid * UPW

    pltpu.sync_copy(user_id_hbm.at[pl.ds(base_u, UPW)], uid_v)
    cu = pltpu.async_copy(user_tab_hbm.at[uid_v], uprows, sem_u)
    pltpu.sync_copy(neg_flat_hbm.at[pl.ds(base_u * N, UPW * N)],
                    ids_all.at[pl.ds(0, UPW * N)])
    cu.wait()

    rows_bufs = (rows0, rows1)
    sems = ((s0a, s0b), (s1a, s1b))

    def issue_gather(u, phase):
        rb = rows_bufs[phase]
        sa, sb = sems[phase]
        pltpu.async_copy(item_tab_hbm.at[ids_all.at[pl.ds(u * N, C1)]],
                         rb.at[pl.ds(0, C1)], sa)
        pltpu.async_copy(item_tab_hbm.at[ids_all.at[pl.ds(u * N + C1, C2)]],
                         rb.at[pl.ds(C1, C2)], sb)

    def wait_gather(u, phase):
        rb = rows_bufs[phase]
        sa, sb = sems[phase]
        pltpu.make_async_copy(item_tab_hbm.at[ids_all.at[pl.ds(u * N, C1)]],
                              rb.at[pl.ds(0, C1)], sa).wait()
        pltpu.make_async_copy(item_tab_hbm.at[ids_all.at[pl.ds(u * N + C1, C2)]],
                              rb.at[pl.ds(C1, C2)], sb).wait()

    issue_gather(0, 0)

    lanes = lax.iota(jnp.int32, L)
    n_vecs = [lanes + jnp.int32(g * L) for g in range(NG)]
    lane_lt8 = lanes < 8

    def do_user(u, phase):
        rows_v = rows_bufs[phase]
        wait_gather(u, phase)

        @pl.when(u + 1 < UPW)
        def _():
            issue_gather(u + 1, 1 - phase)

        u_splat = jnp.full((L,), u, jnp.int32)

        def d_body(dp, accs):
            dp_splat = jnp.full((L,), dp, jnp.int32)
            uw = plsc.load_gather(uprows, [u_splat, dp_splat])
            ue, uo = _bf16_split(uw)
            out = []
            for g, acc in enumerate(accs):
                w = plsc.load_gather(rows_v, [n_vecs[g], dp_splat])
                e, o = _bf16_split(w)
                out.append(acc + e * ue + o * uo)
            return tuple(out)

        accs = lax.fori_loop(
            0, DP, d_body, tuple(jnp.zeros((L,), jnp.float32) for _ in range(NG)))

        cur_max = jnp.full((L,), NEG_INF)
        cur_n = jnp.full((L,), 2147483647, jnp.int32)
        cur_id = jnp.zeros((L,), jnp.int32)
        for g in range(NG):
            s_g = accs[g]
            if g == NG - 1:
                s_g = jnp.where(lane_lt8, s_g, NEG_INF)
            ids_g = ids_all[pl.ds(u * N + g * L, L)]
            upd = s_g > cur_max
            cur_max = jnp.where(upd, s_g, cur_max)
            cur_n = jnp.where(upd, n_vecs[g], cur_n)
            cur_id = jnp.where(upd, ids_g, cur_id)
            scores_all[pl.ds(u * N + g * L, L)] = s_g

        m = jnp.max(cur_max)
        big = jnp.int32(2147483647)
        n_sel = jnp.min(jnp.where(cur_max == m, cur_n, big))
        id_sel = jnp.min(jnp.where(cur_n == n_sel, cur_id, big))
        plsc.store_scatter(selid_v, [u_splat],
                           jnp.full((L,), id_sel, jnp.int32),
                           mask=lanes == 0)

    def pair_body(i, _):
        do_user(2 * i, 0)
        do_user(2 * i + 1, 1)
        return _

    lax.fori_loop(0, UPW // 2, pair_body, None)

    pltpu.sync_copy(scores_all.at[pl.ds(0, UPW * N)],
                    scores_out_hbm.at[pl.ds(base_u * N, UPW * N)])
    pltpu.sync_copy(selid_v, sel_out_hbm.at[pl.ds(base_u, UPW)])


@jax.jit
def _run(user_id, neg_flat, user_packed, item_packed):
    mesh = plsc.VectorSubcoreMesh(core_axis_name="c", subcore_axis_name="s",
                                  num_cores=NC, num_subcores=NS)
    f = pl.kernel(
        _sc_body,
        out_type=(
            jax.ShapeDtypeStruct((B * N,), jnp.float32),
            jax.ShapeDtypeStruct((B,), jnp.int32),
        ),
        mesh=mesh,
        compiler_params=pltpu.CompilerParams(needs_layout_passes=False,
                                             use_tc_tiling_on_sc=False),
        scratch_types=(
            pltpu.VMEM((UPW,), jnp.int32),
            pltpu.VMEM((UPW, DP), jnp.int32),
            pltpu.VMEM((UPW * N + L,), jnp.int32),
            pltpu.VMEM((NG * L, DPP), jnp.int32),
            pltpu.VMEM((NG * L, DPP), jnp.int32),
            pltpu.VMEM((UPW * N + L,), jnp.float32),
            pltpu.VMEM((UPW,), jnp.int32),
            pltpu.SemaphoreType.DMA,
            pltpu.SemaphoreType.DMA,
            pltpu.SemaphoreType.DMA,
            pltpu.SemaphoreType.DMA,
            pltpu.SemaphoreType.DMA,
        ),
    )
    return f(user_id, neg_flat, user_packed, item_packed)


def _pack_bf16(table, pad=False):
    vocab = table.shape[0]
    packed = lax.bitcast_convert_type(
        table.astype(jnp.bfloat16).reshape(vocab, DP, 2), jnp.int32)
    if pad:
        packed = jnp.pad(packed, ((0, 0), (0, DPP - DP)))
    return packed


def kernel(user_id, neg_item_ids, user_emb_table, item_emb_table):
    scores_flat, sel = _run(user_id.astype(jnp.int32),
                            neg_item_ids.reshape(-1),
                            _pack_bf16(user_emb_table),
                            _pack_bf16(item_emb_table, pad=True))
    return scores_flat.reshape(B, N), sel.reshape(B, 1)


# speedup vs baseline: 1.7837x; 1.1952x over previous
import jax
import jax.numpy as jnp
import numpy as np
from jax import lax
from jax.experimental import pallas as pl
from jax.experimental.pallas import tpu as pltpu
from jax.experimental.pallas import tpu_sc as plsc

B = 4096
N = 200
D = 128
DP = D // 2
NC = 2
NS = 16
L = 16
NW = NC * NS
UPW = B // NW
NG = 13
C1 = 128
C2 = N - C1
NEG_INF = float("-inf")
HI_MASK = np.int32(np.uint32(0xFFFF0000))


def _bf16_split(w):
    even = plsc.bitcast(w << jnp.int32(16), jnp.float32)
    odd = plsc.bitcast(w & HI_MASK, jnp.float32)
    return even, odd


def _sc_body(user_id_hbm, neg_flat_hbm, user_tab_hbm, item_tab_hbm,
             scores_out_hbm, sel_out_hbm,
             uid_v, uprows, ids_all, rows0, rows1, scores_all, selid_v,
             sem_u, s0a, s0b, s1a, s1b):
    wid = lax.axis_index("s") * NC + lax.axis_index("c")
    base_u = wid * UPW

    pltpu.sync_copy(user_id_hbm.at[pl.ds(base_u, UPW)], uid_v)
    cu = pltpu.async_copy(user_tab_hbm.at[uid_v], uprows, sem_u)
    pltpu.sync_copy(neg_flat_hbm.at[pl.ds(base_u * N, UPW * N)],
                    ids_all.at[pl.ds(0, UPW * N)])
    cu.wait()

    rows_bufs = (rows0, rows1)
    sems = ((s0a, s0b), (s1a, s1b))

    def issue_gather(u, phase):
        rb = rows_bufs[phase]
        sa, sb = sems[phase]
        pltpu.async_copy(item_tab_hbm.at[ids_all.at[pl.ds(u * N, C1)]],
                         rb.at[pl.ds(0, C1)], sa)
        pltpu.async_copy(item_tab_hbm.at[ids_all.at[pl.ds(u * N + C1, C2)]],
                         rb.at[pl.ds(C1, C2)], sb)

    def wait_gather(u, phase):
        rb = rows_bufs[phase]
        sa, sb = sems[phase]
        pltpu.make_async_copy(item_tab_hbm.at[ids_all.at[pl.ds(u * N, C1)]],
                              rb.at[pl.ds(0, C1)], sa).wait()
        pltpu.make_async_copy(item_tab_hbm.at[ids_all.at[pl.ds(u * N + C1, C2)]],
                              rb.at[pl.ds(C1, C2)], sb).wait()

    issue_gather(0, 0)

    lanes = lax.iota(jnp.int32, L)
    lane_lt8 = lanes < 8

    def do_user(u, phase):
        rows_v = rows_bufs[phase]
        wait_gather(u, phase)

        @pl.when(u + 1 < UPW)
        def _():
            issue_gather(u + 1, 1 - phase)

        u_splat = jnp.full((L,), u, jnp.int32)

        ue, uo = [], []
        for k in range(DP // L):
            e, o = _bf16_split(uprows[u, pl.ds(k * L, L)])
            ue.append(e)
            uo.append(o)

        lane15 = jnp.full((L,), L - 1, jnp.int32)

        def g_body(g, carry):
            cur_max, cur_n, cur_id = carry
            s_g = jnp.zeros((L,), jnp.float32)
            for l in range(L):
                n = g * L + l
                acc = None
                for k in range(DP // L):
                    e, o = _bf16_split(rows_v[n, pl.ds(k * L, L)])
                    t = e * ue[k] + o * uo[k]
                    acc = t if acc is None else acc + t
                tot = plsc.cumsum(acc)[lane15]
                s_g = jnp.where(lanes == jnp.int32(l), tot, s_g)
            g_is_last = g == NG - 1
            s_g = jnp.where(lane_lt8 | jnp.logical_not(g_is_last), s_g, NEG_INF)
            n_vec_g = lanes + g * L
            ids_g = ids_all[pl.ds(u * N + g * L, L)]
            upd = s_g > cur_max
            cur_max = jnp.where(upd, s_g, cur_max)
            cur_n = jnp.where(upd, n_vec_g, cur_n)
            cur_id = jnp.where(upd, ids_g, cur_id)
            scores_all[pl.ds(u * N + g * L, L)] = s_g
            return cur_max, cur_n, cur_id

        cur_max, cur_n, cur_id = lax.fori_loop(
            0, NG, g_body,
            (jnp.full((L,), NEG_INF),
             jnp.full((L,), 2147483647, jnp.int32),
             jnp.zeros((L,), jnp.int32)))

        m = jnp.max(cur_max)
        big = jnp.int32(2147483647)
        n_sel = jnp.min(jnp.where(cur_max == m, cur_n, big))
        id_sel = jnp.min(jnp.where(cur_n == n_sel, cur_id, big))
        plsc.store_scatter(selid_v, [u_splat],
                           jnp.full((L,), id_sel, jnp.int32),
                           mask=lanes == 0)

    def pair_body(i, _):
        do_user(2 * i, 0)
        do_user(2 * i + 1, 1)
        return _

    lax.fori_loop(0, UPW // 2, pair_body, None)

    pltpu.sync_copy(scores_all.at[pl.ds(0, UPW * N)],
                    scores_out_hbm.at[pl.ds(base_u * N, UPW * N)])
    pltpu.sync_copy(selid_v, sel_out_hbm.at[pl.ds(base_u, UPW)])


@jax.jit
def _run(user_id, neg_flat, user_packed, item_packed):
    mesh = plsc.VectorSubcoreMesh(core_axis_name="c", subcore_axis_name="s",
                                  num_cores=NC, num_subcores=NS)
    f = pl.kernel(
        _sc_body,
        out_type=(
            jax.ShapeDtypeStruct((B * N,), jnp.float32),
            jax.ShapeDtypeStruct((B,), jnp.int32),
        ),
        mesh=mesh,
        compiler_params=pltpu.CompilerParams(needs_layout_passes=False,
                                             use_tc_tiling_on_sc=False),
        scratch_types=(
            pltpu.VMEM((UPW,), jnp.int32),
            pltpu.VMEM((UPW, DP), jnp.int32),
            pltpu.VMEM((UPW * N + L,), jnp.int32),
            pltpu.VMEM((NG * L, DP), jnp.int32),
            pltpu.VMEM((NG * L, DP), jnp.int32),
            pltpu.VMEM((UPW * N + L,), jnp.float32),
            pltpu.VMEM((UPW,), jnp.int32),
            pltpu.SemaphoreType.DMA,
            pltpu.SemaphoreType.DMA,
            pltpu.SemaphoreType.DMA,
            pltpu.SemaphoreType.DMA,
            pltpu.SemaphoreType.DMA,
        ),
    )
    return f(user_id, neg_flat, user_packed, item_packed)


def _pack_bf16(table):
    vocab = table.shape[0]
    return lax.bitcast_convert_type(
        table.astype(jnp.bfloat16).reshape(vocab, DP, 2), jnp.int32)


def kernel(user_id, neg_item_ids, user_emb_table, item_emb_table):
    scores_flat, sel = _run(user_id.astype(jnp.int32),
                            neg_item_ids.reshape(-1),
                            _pack_bf16(user_emb_table),
                            _pack_bf16(item_emb_table))
    return scores_flat.reshape(B, N), sel.reshape(B, 1)


# speedup vs baseline: 6.2690x; 3.5146x over previous
import jax
import jax.numpy as jnp
import numpy as np
from jax import lax
from jax.experimental import pallas as pl
from jax.experimental.pallas import tpu as pltpu
from jax.experimental.pallas import tpu_sc as plsc

B = 4096
N = 200
D = 128
DP = D // 2
NC = 2
NS = 16
L = 16
NW = NC * NS
UPW = B // NW
NG = 13
C1 = 128
C2 = N - C1
NEG_INF = float("-inf")
HI_MASK = np.int32(np.uint32(0xFFFF0000))


def _bf16_split(w):
    even = plsc.bitcast(w << jnp.int32(16), jnp.float32)
    odd = plsc.bitcast(w & HI_MASK, jnp.float32)
    return even, odd


def _round_bf16(x):
    p = plsc.bitcast(x, jnp.int32)
    p = p + jnp.int32(0x7FFF) + ((p >> jnp.int32(16)) & jnp.int32(1))
    return plsc.bitcast(p & HI_MASK, jnp.float32)


def _sc_body(user_id_hbm, neg_flat_hbm, user_tab_hbm, item_tab_hbm,
             scores_out_hbm, sel_out_hbm,
             uid_v, uprows, ids_all, rows0, rows1, scores_all, selid_v,
             sem_u, s0a, s0b, s1a, s1b):
    wid = lax.axis_index("s") * NC + lax.axis_index("c")
    base_u = wid * UPW

    pltpu.sync_copy(user_id_hbm.at[pl.ds(base_u, UPW)], uid_v)
    cu = pltpu.async_copy(user_tab_hbm.at[uid_v], uprows, sem_u)
    pltpu.sync_copy(neg_flat_hbm.at[pl.ds(base_u * N, UPW * N)],
                    ids_all.at[pl.ds(0, UPW * N)])
    cu.wait()

    rows_bufs = (rows0, rows1)
    sems = ((s0a, s0b), (s1a, s1b))

    def issue_gather(u, phase):
        rb = rows_bufs[phase]
        sa, sb = sems[phase]
        pltpu.async_copy(item_tab_hbm.at[ids_all.at[pl.ds(u * N, C1)]],
                         rb.at[pl.ds(0, C1)], sa)
        pltpu.async_copy(item_tab_hbm.at[ids_all.at[pl.ds(u * N + C1, C2)]],
                         rb.at[pl.ds(C1, C2)], sb)

    def wait_gather(u, phase):
        rb = rows_bufs[phase]
        sa, sb = sems[phase]
        pltpu.make_async_copy(item_tab_hbm.at[ids_all.at[pl.ds(u * N, C1)]],
                              rb.at[pl.ds(0, C1)], sa).wait()
        pltpu.make_async_copy(item_tab_hbm.at[ids_all.at[pl.ds(u * N + C1, C2)]],
                              rb.at[pl.ds(C1, C2)], sb).wait()

    issue_gather(0, 0)

    lanes = lax.iota(jnp.int32, L)
    lane_lt8 = lanes < 8

    def do_user(u, phase):
        rows_v = rows_bufs[phase]
        wait_gather(u, phase)

        @pl.when(u + 1 < UPW)
        def _():
            issue_gather(u + 1, 1 - phase)

        u_splat = jnp.full((L,), u, jnp.int32)

        uch = [_round_bf16(uprows[u, pl.ds(k * L, L)]) for k in range(D // L)]
        ue = uch[:DP // L]
        uo = uch[DP // L:]

        lane15 = jnp.full((L,), L - 1, jnp.int32)

        def g_body(g, carry):
            cur_max, cur_n, cur_id = carry
            s_g = jnp.zeros((L,), jnp.float32)
            for l in range(L):
                n = g * L + l
                acc = None
                for k in range(DP // L):
                    e, o = _bf16_split(rows_v[n, pl.ds(k * L, L)])
                    t = e * ue[k] + o * uo[k]
                    acc = t if acc is None else acc + t
                tot = plsc.cumsum(acc)[lane15]
                s_g = jnp.where(lanes == jnp.int32(l), tot, s_g)
            g_is_last = g == NG - 1
            s_g = jnp.where(lane_lt8 | jnp.logical_not(g_is_last), s_g, NEG_INF)
            n_vec_g = lanes + g * L
            ids_g = ids_all[pl.ds(u * N + g * L, L)]
            upd = s_g > cur_max
            cur_max = jnp.where(upd, s_g, cur_max)
            cur_n = jnp.where(upd, n_vec_g, cur_n)
            cur_id = jnp.where(upd, ids_g, cur_id)
            scores_all[pl.ds(u * N + g * L, L)] = s_g
            return cur_max, cur_n, cur_id

        cur_max, cur_n, cur_id = lax.fori_loop(
            0, NG, g_body,
            (jnp.full((L,), NEG_INF),
             jnp.full((L,), 2147483647, jnp.int32),
             jnp.zeros((L,), jnp.int32)))

        m = jnp.max(cur_max)
        big = jnp.int32(2147483647)
        n_sel = jnp.min(jnp.where(cur_max == m, cur_n, big))
        id_sel = jnp.min(jnp.where(cur_n == n_sel, cur_id, big))
        plsc.store_scatter(selid_v, [u_splat],
                           jnp.full((L,), id_sel, jnp.int32),
                           mask=lanes == 0)

    def pair_body(i, _):
        do_user(2 * i, 0)
        do_user(2 * i + 1, 1)
        return _

    lax.fori_loop(0, UPW // 2, pair_body, None)

    pltpu.sync_copy(scores_all.at[pl.ds(0, UPW * N)],
                    scores_out_hbm.at[pl.ds(base_u * N, UPW * N)])
    pltpu.sync_copy(selid_v, sel_out_hbm.at[pl.ds(base_u, UPW)])


@jax.jit
def _run(user_id, neg_flat, user_packed, item_packed):
    mesh = plsc.VectorSubcoreMesh(core_axis_name="c", subcore_axis_name="s",
                                  num_cores=NC, num_subcores=NS)
    f = pl.kernel(
        _sc_body,
        out_type=(
            jax.ShapeDtypeStruct((B * N,), jnp.float32),
            jax.ShapeDtypeStruct((B,), jnp.int32),
        ),
        mesh=mesh,
        compiler_params=pltpu.CompilerParams(needs_layout_passes=False,
                                             use_tc_tiling_on_sc=False),
        scratch_types=(
            pltpu.VMEM((UPW,), jnp.int32),
            pltpu.VMEM((UPW, D), jnp.float32),
            pltpu.VMEM((UPW * N + L,), jnp.int32),
            pltpu.VMEM((NG * L, DP), jnp.int32),
            pltpu.VMEM((NG * L, DP), jnp.int32),
            pltpu.VMEM((UPW * N + L,), jnp.float32),
            pltpu.VMEM((UPW,), jnp.int32),
            pltpu.SemaphoreType.DMA,
            pltpu.SemaphoreType.DMA,
            pltpu.SemaphoreType.DMA,
            pltpu.SemaphoreType.DMA,
            pltpu.SemaphoreType.DMA,
        ),
    )
    return f(user_id, neg_flat, user_packed, item_packed)


def _pack_body(tab_ref, out_ref):
    p = lax.bitcast_convert_type(tab_ref[...], jnp.int32)
    p = p + jnp.int32(0x7FFF) + ((p >> jnp.int32(16)) & jnp.int32(1))
    lo = (p[:, :DP] >> jnp.int32(16)) & jnp.int32(0xFFFF)
    hi = p[:, DP:] & HI_MASK
    out_ref[...] = lo | hi


def _pack_item_table(table):
    vocab = table.shape[0]
    rows = 1000
    return pl.pallas_call(
        _pack_body,
        grid=(vocab // rows,),
        in_specs=[pl.BlockSpec((rows, D), lambda i: (i, 0))],
        out_specs=pl.BlockSpec((rows, DP), lambda i: (i, 0)),
        out_shape=jax.ShapeDtypeStruct((vocab, DP), jnp.int32),
    )(table)


def kernel(user_id, neg_item_ids, user_emb_table, item_emb_table):
    scores_flat, sel = _run(user_id.astype(jnp.int32),
                            neg_item_ids.reshape(-1),
                            user_emb_table,
                            _pack_item_table(item_emb_table))
    return scores_flat.reshape(B, N), sel.reshape(B, 1)


# speedup vs baseline: 6.9871x; 1.1145x over previous
import jax
import jax.numpy as jnp
import numpy as np
from jax import lax
from jax.experimental import pallas as pl
from jax.experimental.pallas import tpu as pltpu
from jax.experimental.pallas import tpu_sc as plsc

B = 4096
N = 200
D = 128
DP = D // 2
NC = 2
NS = 16
L = 16
NW = NC * NS
UPW = B // NW
NG = 13
C1 = 128
C2 = N - C1
NEG_INF = float("-inf")
HI_MASK = np.int32(np.uint32(0xFFFF0000))


def _bf16_split(w):
    even = plsc.bitcast(w << jnp.int32(16), jnp.float32)
    odd = plsc.bitcast(w & HI_MASK, jnp.float32)
    return even, odd


def _round_bf16(x):
    p = plsc.bitcast(x, jnp.int32)
    p = p + jnp.int32(0x7FFF) + ((p >> jnp.int32(16)) & jnp.int32(1))
    return plsc.bitcast(p & HI_MASK, jnp.float32)


def _sc_body(user_id_hbm, neg_hbm, user_tab_hbm, item_tab_hbm,
             scores_out_hbm, sel_out_hbm,
             uid_v, uprows, ids2d, rows0, rows1, scores2d, selid_v,
             sem_u, s0a, s0b, s1a, s1b):
    wid = lax.axis_index("s") * NC + lax.axis_index("c")
    base_u = wid * UPW

    pltpu.sync_copy(user_id_hbm.at[pl.ds(base_u, UPW)], uid_v)
    cu = pltpu.async_copy(user_tab_hbm.at[uid_v], uprows, sem_u)
    pltpu.sync_copy(neg_hbm.at[pl.ds(base_u, UPW)], ids2d)
    cu.wait()

    rows_bufs = (rows0, rows1)
    sems = ((s0a, s0b), (s1a, s1b))

    def issue_gather(u, phase):
        rb = rows_bufs[phase]
        sa, sb = sems[phase]
        pltpu.async_copy(item_tab_hbm.at[ids2d.at[u, pl.ds(0, C1)]],
                         rb.at[pl.ds(0, C1)], sa)
        pltpu.async_copy(item_tab_hbm.at[ids2d.at[u, pl.ds(C1, C2)]],
                         rb.at[pl.ds(C1, C2)], sb)

    def wait_gather(u, phase):
        rb = rows_bufs[phase]
        sa, sb = sems[phase]
        pltpu.make_async_copy(item_tab_hbm.at[ids2d.at[u, pl.ds(0, C1)]],
                              rb.at[pl.ds(0, C1)], sa).wait()
        pltpu.make_async_copy(item_tab_hbm.at[ids2d.at[u, pl.ds(C1, C2)]],
                              rb.at[pl.ds(C1, C2)], sb).wait()

    issue_gather(0, 0)

    lanes = lax.iota(jnp.int32, L)
    lane_lt8 = lanes < 8

    def do_user(u, phase):
        rows_v = rows_bufs[phase]
        wait_gather(u, phase)

        @pl.when(u + 1 < UPW)
        def _():
            issue_gather(u + 1, 1 - phase)

        u_splat = jnp.full((L,), u, jnp.int32)

        uch = [_round_bf16(uprows[u, pl.ds(k * L, L)]) for k in range(D // L)]
        ue = uch[:DP // L]
        uo = uch[DP // L:]

        lane15 = jnp.full((L,), L - 1, jnp.int32)

        def g_body(g, carry):
            cur_max, cur_n, cur_id = carry
            s_g = jnp.zeros((L,), jnp.float32)
            for l in range(L):
                n = g * L + l
                acc = None
                for k in range(DP // L):
                    e, o = _bf16_split(rows_v[n, pl.ds(k * L, L)])
                    t = e * ue[k] + o * uo[k]
                    acc = t if acc is None else acc + t
                tot = plsc.cumsum(acc)[lane15]
                s_g = jnp.where(lanes == jnp.int32(l), tot, s_g)
            g_is_last = g == NG - 1
            s_g = jnp.where(lane_lt8 | jnp.logical_not(g_is_last), s_g, NEG_INF)
            n_vec_g = lanes + g * L
            ids_g = lax.cond(
                g_is_last,
                lambda: ids2d[u, pl.ds(N - L, L)][jnp.minimum(lanes + 8, L - 1)],
                lambda: ids2d[u, pl.ds(jnp.minimum(g, NG - 2) * L, L)])
            upd = s_g > cur_max
            cur_max = jnp.where(upd, s_g, cur_max)
            cur_n = jnp.where(upd, n_vec_g, cur_n)
            cur_id = jnp.where(upd, ids_g, cur_id)
            scores2d[u, pl.ds(g * L, L)] = s_g
            return cur_max, cur_n, cur_id

        cur_max, cur_n, cur_id = lax.fori_loop(
            0, NG, g_body,
            (jnp.full((L,), NEG_INF),
             jnp.full((L,), 2147483647, jnp.int32),
             jnp.zeros((L,), jnp.int32)))

        m = jnp.max(cur_max)
        big = jnp.int32(2147483647)
        n_sel = jnp.min(jnp.where(cur_max == m, cur_n, big))
        id_sel = jnp.min(jnp.where(cur_n == n_sel, cur_id, big))
        plsc.store_scatter(selid_v, [u_splat, jnp.zeros((L,), jnp.int32)],
                           jnp.full((L,), id_sel, jnp.int32),
                           mask=lanes == 0)

    def pair_body(i, _):
        do_user(2 * i, 0)
        do_user(2 * i + 1, 1)
        return _

    lax.fori_loop(0, UPW // 2, pair_body, None)

    pltpu.sync_copy(scores2d.at[:, pl.ds(0, N)],
                    scores_out_hbm.at[pl.ds(base_u, UPW)])
    pltpu.sync_copy(selid_v, sel_out_hbm.at[pl.ds(base_u, UPW)])


@jax.jit
def _run(user_id, neg_flat, user_packed, item_packed):
    mesh = plsc.VectorSubcoreMesh(core_axis_name="c", subcore_axis_name="s",
                                  num_cores=NC, num_subcores=NS)
    f = pl.kernel(
        _sc_body,
        out_type=(
            jax.ShapeDtypeStruct((B, N), jnp.float32),
            jax.ShapeDtypeStruct((B, 1), jnp.int32),
        ),
        mesh=mesh,
        compiler_params=pltpu.CompilerParams(needs_layout_passes=False,
                                             use_tc_tiling_on_sc=False),
        scratch_types=(
            pltpu.VMEM((UPW,), jnp.int32),
            pltpu.VMEM((UPW, D), jnp.float32),
            pltpu.VMEM((UPW, N), jnp.int32),
            pltpu.VMEM((NG * L, DP), jnp.int32),
            pltpu.VMEM((NG * L, DP), jnp.int32),
            pltpu.VMEM((UPW, NG * L), jnp.float32),
            pltpu.VMEM((UPW, 1), jnp.int32),
            pltpu.SemaphoreType.DMA,
            pltpu.SemaphoreType.DMA,
            pltpu.SemaphoreType.DMA,
            pltpu.SemaphoreType.DMA,
            pltpu.SemaphoreType.DMA,
        ),
    )
    return f(user_id, neg_flat, user_packed, item_packed)


def _pack_body(tab_ref, out_ref):
    p = lax.bitcast_convert_type(tab_ref[...], jnp.int32)
    p = p + jnp.int32(0x7FFF) + ((p >> jnp.int32(16)) & jnp.int32(1))
    lo = (p[:, :DP] >> jnp.int32(16)) & jnp.int32(0xFFFF)
    hi = p[:, DP:] & HI_MASK
    out_ref[...] = lo | hi


def _pack_item_table(table):
    vocab = table.shape[0]
    rows = 4000
    return pl.pallas_call(
        _pack_body,
        grid=(vocab // rows,),
        in_specs=[pl.BlockSpec((rows, D), lambda i: (i, 0))],
        out_specs=pl.BlockSpec((rows, DP), lambda i: (i, 0)),
        out_shape=jax.ShapeDtypeStruct((vocab, DP), jnp.int32),
    )(table)


def kernel(user_id, neg_item_ids, user_emb_table, item_emb_table):
    return _run(user_id.astype(jnp.int32), neg_item_ids,
                user_emb_table, _pack_item_table(item_emb_table))


# speedup vs baseline: 8.2306x; 1.1780x over previous
import jax
import jax.numpy as jnp
import numpy as np
from jax import lax
from jax.experimental import pallas as pl
from jax.experimental.pallas import tpu as pltpu
from jax.experimental.pallas import tpu_sc as plsc

B = 4096
VOCAB = 100000
N = 200
D = 128
DP = D // 2
NC = 2
NS = 16
L = 16
NW = NC * NS
UPW = B // NW
NG = 13
C1 = 128
C2 = N - C1
NEG_INF = float("-inf")
HI_MASK = np.int32(np.uint32(0xFFFF0000))


def _bf16_split(w):
    even = plsc.bitcast(w << jnp.int32(16), jnp.float32)
    odd = plsc.bitcast(w & HI_MASK, jnp.float32)
    return even, odd


def _round_bf16(x):
    p = plsc.bitcast(x, jnp.int32)
    p = p + jnp.int32(0x7FFF) + ((p >> jnp.int32(16)) & jnp.int32(1))
    return plsc.bitcast(p & HI_MASK, jnp.float32)


def _sc_body(user_id_hbm, neg_hbm, user_tab_hbm, item_tab_hbm,
             scores_out_hbm, sel_out_hbm,
             uid_v, uprows, ids2d, rows0, rows1, scores2d, selid_v,
             sem_u, s0a, s0b, s1a, s1b):
    wid = lax.axis_index("s") * NC + lax.axis_index("c")
    base_u = wid * UPW

    pltpu.sync_copy(user_id_hbm.at[pl.ds(base_u, UPW)], uid_v)
    cu = pltpu.async_copy(user_tab_hbm.at[uid_v], uprows, sem_u)
    pltpu.sync_copy(neg_hbm.at[pl.ds(base_u, UPW)], ids2d)
    cu.wait()

    lanes0 = lax.iota(jnp.int32, L)
    v2 = jnp.int32(VOCAB // 2)

    def _to_packed_row(i):
        return (i << jnp.int32(1)) - jnp.where(i >= v2, v2 * 2 - 1, 0)

    def xform_user(ui, _):
        for g in range(NG - 1):
            ids2d[ui, pl.ds(g * L, L)] = _to_packed_row(ids2d[ui, pl.ds(g * L, L)])
        w = _to_packed_row(ids2d[ui, pl.ds(N - L, L)])
        plsc.store_scatter(ids2d, [jnp.full((L,), ui, jnp.int32), lanes0 + (N - L)],
                           w, mask=lanes0 >= 8)
        return _
    lax.fori_loop(0, UPW, xform_user, None)

    rows_bufs = (rows0, rows1)
    sems = ((s0a, s0b), (s1a, s1b))

    def issue_gather(u, phase):
        rb = rows_bufs[phase]
        sa, sb = sems[phase]
        pltpu.async_copy(item_tab_hbm.at[ids2d.at[u, pl.ds(0, C1)]],
                         rb.at[pl.ds(0, C1)], sa)
        pltpu.async_copy(item_tab_hbm.at[ids2d.at[u, pl.ds(C1, C2)]],
                         rb.at[pl.ds(C1, C2)], sb)

    def wait_gather(u, phase):
        rb = rows_bufs[phase]
        sa, sb = sems[phase]
        pltpu.make_async_copy(item_tab_hbm.at[ids2d.at[u, pl.ds(0, C1)]],
                              rb.at[pl.ds(0, C1)], sa).wait()
        pltpu.make_async_copy(item_tab_hbm.at[ids2d.at[u, pl.ds(C1, C2)]],
                              rb.at[pl.ds(C1, C2)], sb).wait()

    issue_gather(0, 0)

    lanes = lax.iota(jnp.int32, L)
    lane_lt8 = lanes < 8

    def do_user(u, phase):
        rows_v = rows_bufs[phase]
        wait_gather(u, phase)

        @pl.when(u + 1 < UPW)
        def _():
            issue_gather(u + 1, 1 - phase)

        u_splat = jnp.full((L,), u, jnp.int32)

        uch = [_round_bf16(uprows[u, pl.ds(k * L, L)]) for k in range(D // L)]
        ue = uch[:DP // L]
        uo = uch[DP // L:]

        lane15 = jnp.full((L,), L - 1, jnp.int32)

        def g_body(g, carry):
            cur_max, cur_n, cur_id = carry
            s_g = jnp.zeros((L,), jnp.float32)
            for l in range(L):
                n = g * L + l
                acc = None
                for k in range(DP // L):
                    e, o = _bf16_split(rows_v[n, pl.ds(k * L, L)])
                    t = e * ue[k] + o * uo[k]
                    acc = t if acc is None else acc + t
                tot = plsc.cumsum(acc)[lane15]
                s_g = jnp.where(lanes == jnp.int32(l), tot, s_g)
            g_is_last = g == NG - 1
            s_g = jnp.where(lane_lt8 | jnp.logical_not(g_is_last), s_g, NEG_INF)
            n_vec_g = lanes + g * L
            ids_g = lax.cond(
                g_is_last,
                lambda: ids2d[u, pl.ds(N - L, L)][jnp.minimum(lanes + 8, L - 1)],
                lambda: ids2d[u, pl.ds(jnp.minimum(g, NG - 2) * L, L)])
            upd = s_g > cur_max
            cur_max = jnp.where(upd, s_g, cur_max)
            cur_n = jnp.where(upd, n_vec_g, cur_n)
            cur_id = jnp.where(upd, ids_g, cur_id)
            scores2d[u, pl.ds(g * L, L)] = s_g
            return cur_max, cur_n, cur_id

        cur_max, cur_n, cur_id = lax.fori_loop(
            0, NG, g_body,
            (jnp.full((L,), NEG_INF),
             jnp.full((L,), 2147483647, jnp.int32),
             jnp.zeros((L,), jnp.int32)))

        m = jnp.max(cur_max)
        big = jnp.int32(2147483647)
        n_sel = jnp.min(jnp.where(cur_max == m, cur_n, big))
        id_sel_t = jnp.min(jnp.where(cur_n == n_sel, cur_id, big))
        id_sel = (id_sel_t >> jnp.int32(1)) + jnp.where(
            (id_sel_t & jnp.int32(1)) == 1, v2, 0)
        plsc.store_scatter(selid_v, [u_splat, jnp.zeros((L,), jnp.int32)],
                           jnp.full((L,), id_sel, jnp.int32),
                           mask=lanes == 0)

    def pair_body(i, _):
        do_user(2 * i, 0)
        do_user(2 * i + 1, 1)
        return _

    lax.fori_loop(0, UPW // 2, pair_body, None)

    pltpu.sync_copy(scores2d.at[:, pl.ds(0, N)],
                    scores_out_hbm.at[pl.ds(base_u, UPW)])
    pltpu.sync_copy(selid_v, sel_out_hbm.at[pl.ds(base_u, UPW)])


@jax.jit
def _run(user_id, neg_flat, user_packed, item_packed):
    mesh = plsc.VectorSubcoreMesh(core_axis_name="c", subcore_axis_name="s",
                                  num_cores=NC, num_subcores=NS)
    f = pl.kernel(
        _sc_body,
        out_type=(
            jax.ShapeDtypeStruct((B, N), jnp.float32),
            jax.ShapeDtypeStruct((B, 1), jnp.int32),
        ),
        mesh=mesh,
        compiler_params=pltpu.CompilerParams(needs_layout_passes=False,
                                             use_tc_tiling_on_sc=False),
        scratch_types=(
            pltpu.VMEM((UPW,), jnp.int32),
            pltpu.VMEM((UPW, D), jnp.float32),
            pltpu.VMEM((UPW, N), jnp.int32),
            pltpu.VMEM((NG * L, DP), jnp.int32),
            pltpu.VMEM((NG * L, DP), jnp.int32),
            pltpu.VMEM((UPW, NG * L), jnp.float32),
            pltpu.VMEM((UPW, 1), jnp.int32),
            pltpu.SemaphoreType.DMA,
            pltpu.SemaphoreType.DMA,
            pltpu.SemaphoreType.DMA,
            pltpu.SemaphoreType.DMA,
            pltpu.SemaphoreType.DMA,
        ),
    )
    return f(user_id, neg_flat, user_packed, item_packed)


def _pack_half(p):
    p = p + jnp.int32(0x7FFF) + ((p >> jnp.int32(16)) & jnp.int32(1))
    return ((p[:, :DP] >> jnp.int32(16)) & jnp.int32(0xFFFF)) | (p[:, DP:] & HI_MASK)


def _pack_body(tab1_ref, tab2_ref, out_ref):
    out_ref[:, :DP] = _pack_half(lax.bitcast_convert_type(tab1_ref[...], jnp.int32))
    out_ref[:, DP:] = _pack_half(lax.bitcast_convert_type(tab2_ref[...], jnp.int32))


def _pack_item_table(table):
    vocab = table.shape[0]
    half_blocks = (vocab // 2) // 2000
    rows = 2000
    packed = pl.pallas_call(
        _pack_body,
        grid=(half_blocks,),
        in_specs=[pl.BlockSpec((rows, D), lambda i: (i, 0)),
                  pl.BlockSpec((rows, D), lambda i, hb=half_blocks: (i + hb, 0))],
        out_specs=pl.BlockSpec((rows, D), lambda i: (i, 0)),
        out_shape=jax.ShapeDtypeStruct((vocab // 2, D), jnp.int32),
    )(table, table)
    return packed.reshape(vocab, DP)


def kernel(user_id, neg_item_ids, user_emb_table, item_emb_table):
    return _run(user_id.astype(jnp.int32), neg_item_ids,
                user_emb_table, _pack_item_table(item_emb_table))


# speedup vs baseline: 8.4420x; 1.0257x over previous
import jax
import jax.numpy as jnp
import numpy as np
from jax import lax
from jax.experimental import pallas as pl
from jax.experimental.pallas import tpu as pltpu
from jax.experimental.pallas import tpu_sc as plsc

B = 4096
VOCAB = 100000
N = 200
D = 128
DP = D // 2
NC = 2
NS = 16
L = 16
NW = NC * NS
UPW = B // NW
NG = 13
C1 = 128
C2 = N - C1
NEG_INF = float("-inf")
HI_MASK = np.int32(np.uint32(0xFFFF0000))


def _bf16_split(w):
    even = plsc.bitcast(w << jnp.int32(16), jnp.float32)
    odd = plsc.bitcast(w & HI_MASK, jnp.float32)
    return even, odd


def _round_bf16(x):
    p = plsc.bitcast(x, jnp.int32)
    p = p + jnp.int32(0x7FFF) + ((p >> jnp.int32(16)) & jnp.int32(1))
    return plsc.bitcast(p & HI_MASK, jnp.float32)


def _sc_body(user_id_hbm, neg_hbm, user_tab_hbm, item_tab_hbm,
             scores_out_hbm, sel_out_hbm,
             uid_v, uprows, ids2d, rows0, rows1, scores2d, selid_v,
             sem_u, s0a, s0b, s1a, s1b):
    wid = lax.axis_index("s") * NC + lax.axis_index("c")
    base_u = wid * UPW

    pltpu.sync_copy(user_id_hbm.at[pl.ds(base_u, UPW)], uid_v)
    cu = pltpu.async_copy(user_tab_hbm.at[uid_v], uprows, sem_u)
    pltpu.sync_copy(neg_hbm.at[pl.ds(base_u, UPW)], ids2d)
    cu.wait()

    lanes0 = lax.iota(jnp.int32, L)
    v2 = jnp.int32(VOCAB // 2)

    def _to_packed_row(i):
        return (i << jnp.int32(1)) - jnp.where(i >= v2, v2 * 2 - 1, 0)

    def xform_user(ui, _):
        for g in range(NG - 1):
            ids2d[ui, pl.ds(g * L, L)] = _to_packed_row(ids2d[ui, pl.ds(g * L, L)])
        w = _to_packed_row(ids2d[ui, pl.ds(N - L, L)])
        plsc.store_scatter(ids2d, [jnp.full((L,), ui, jnp.int32), lanes0 + (N - L)],
                           w, mask=lanes0 >= 8)
        return _
    lax.fori_loop(0, UPW, xform_user, None)

    rows_bufs = (rows0, rows1)
    sems = ((s0a, s0b), (s1a, s1b))

    def issue_gather(u, phase):
        rb = rows_bufs[phase]
        sa, sb = sems[phase]
        pltpu.async_copy(item_tab_hbm.at[ids2d.at[u, pl.ds(0, C1)]],
                         rb.at[pl.ds(0, C1)], sa)
        pltpu.async_copy(item_tab_hbm.at[ids2d.at[u, pl.ds(C1, C2)]],
                         rb.at[pl.ds(C1, C2)], sb)

    def wait_gather(u, phase):
        rb = rows_bufs[phase]
        sa, sb = sems[phase]
        pltpu.make_async_copy(item_tab_hbm.at[ids2d.at[u, pl.ds(0, C1)]],
                              rb.at[pl.ds(0, C1)], sa).wait()
        pltpu.make_async_copy(item_tab_hbm.at[ids2d.at[u, pl.ds(C1, C2)]],
                              rb.at[pl.ds(C1, C2)], sb).wait()

    issue_gather(0, 0)

    lanes = lax.iota(jnp.int32, L)
    lane_lt8 = lanes < 8

    def do_user(u, phase):
        rows_v = rows_bufs[phase]
        wait_gather(u, phase)

        @pl.when(u + 1 < UPW)
        def _():
            issue_gather(u + 1, 1 - phase)

        u_splat = jnp.full((L,), u, jnp.int32)

        uch = [_round_bf16(uprows[u, pl.ds(k * L, L)]) for k in range(D // L)]
        ue = uch[:DP // L]
        uo = uch[DP // L:]

        lane15 = jnp.full((L,), L - 1, jnp.int32)

        def g_body(g, carry):
            cur_max, cur_n = carry
            s_g = jnp.zeros((L,), jnp.float32)
            for l in range(L):
                n = g * L + l
                acc = None
                for k in range(DP // L):
                    e, o = _bf16_split(rows_v[n, pl.ds(k * L, L)])
                    t = e * ue[k] + o * uo[k]
                    acc = t if acc is None else acc + t
                tot = plsc.cumsum(acc)[lane15]
                s_g = jnp.where(lanes == jnp.int32(l), tot, s_g)
            g_is_last = g == NG - 1
            s_g = jnp.where(lane_lt8 | jnp.logical_not(g_is_last), s_g, NEG_INF)
            n_vec_g = lanes + g * L
            upd = s_g > cur_max
            cur_max = jnp.where(upd, s_g, cur_max)
            cur_n = jnp.where(upd, n_vec_g, cur_n)
            scores2d[u, pl.ds(g * L, L)] = s_g
            return cur_max, cur_n

        cur_max, cur_n = lax.fori_loop(
            0, NG, g_body,
            (jnp.full((L,), NEG_INF),
             jnp.full((L,), 2147483647, jnp.int32)))

        m = jnp.max(cur_max)
        big = jnp.int32(2147483647)
        n_sel = jnp.min(jnp.where(cur_max == m, cur_n, big))
        id_sel_t = plsc.load_gather(
            ids2d, [u_splat, jnp.full((L,), n_sel, jnp.int32)])
        id_sel = (id_sel_t >> jnp.int32(1)) + jnp.where(
            (id_sel_t & jnp.int32(1)) == 1, v2, 0)
        plsc.store_scatter(selid_v, [u_splat, jnp.zeros((L,), jnp.int32)],
                           id_sel, mask=lanes == 0)

    def pair_body(i, _):
        do_user(2 * i, 0)
        do_user(2 * i + 1, 1)
        return _

    lax.fori_loop(0, UPW // 2, pair_body, None)

    pltpu.sync_copy(scores2d.at[:, pl.ds(0, N)],
                    scores_out_hbm.at[pl.ds(base_u, UPW)])
    pltpu.sync_copy(selid_v, sel_out_hbm.at[pl.ds(base_u, UPW)])


@jax.jit
def _run(user_id, neg_flat, user_packed, item_packed):
    mesh = plsc.VectorSubcoreMesh(core_axis_name="c", subcore_axis_name="s",
                                  num_cores=NC, num_subcores=NS)
    f = pl.kernel(
        _sc_body,
        out_type=(
            jax.ShapeDtypeStruct((B, N), jnp.float32),
            jax.ShapeDtypeStruct((B, 1), jnp.int32),
        ),
        mesh=mesh,
        compiler_params=pltpu.CompilerParams(needs_layout_passes=False,
                                             use_tc_tiling_on_sc=False),
        scratch_types=(
            pltpu.VMEM((UPW,), jnp.int32),
            pltpu.VMEM((UPW, D), jnp.float32),
            pltpu.VMEM((UPW, N), jnp.int32),
            pltpu.VMEM((NG * L, DP), jnp.int32),
            pltpu.VMEM((NG * L, DP), jnp.int32),
            pltpu.VMEM((UPW, NG * L), jnp.float32),
            pltpu.VMEM((UPW, 1), jnp.int32),
            pltpu.SemaphoreType.DMA,
            pltpu.SemaphoreType.DMA,
            pltpu.SemaphoreType.DMA,
            pltpu.SemaphoreType.DMA,
            pltpu.SemaphoreType.DMA,
        ),
    )
    return f(user_id, neg_flat, user_packed, item_packed)


def _pack_half(p):
    p = p + jnp.int32(0x7FFF) + ((p >> jnp.int32(16)) & jnp.int32(1))
    return ((p[:, :DP] >> jnp.int32(16)) & jnp.int32(0xFFFF)) | (p[:, DP:] & HI_MASK)


def _pack_body(tab1_ref, tab2_ref, out_ref):
    out_ref[:, :DP] = _pack_half(lax.bitcast_convert_type(tab1_ref[...], jnp.int32))
    out_ref[:, DP:] = _pack_half(lax.bitcast_convert_type(tab2_ref[...], jnp.int32))


def _pack_item_table(table):
    vocab = table.shape[0]
    half_blocks = (vocab // 2) // 2000
    rows = 2000
    packed = pl.pallas_call(
        _pack_body,
        grid=(half_blocks,),
        in_specs=[pl.BlockSpec((rows, D), lambda i: (i, 0)),
                  pl.BlockSpec((rows, D), lambda i, hb=half_blocks: (i + hb, 0))],
        out_specs=pl.BlockSpec((rows, D), lambda i: (i, 0)),
        out_shape=jax.ShapeDtypeStruct((vocab // 2, D), jnp.int32),
    )(table, table)
    return packed.reshape(vocab, DP)


def kernel(user_id, neg_item_ids, user_emb_table, item_emb_table):
    return _run(user_id.astype(jnp.int32), neg_item_ids,
                user_emb_table, _pack_item_table(item_emb_table))
